# baseline (device time: 163453 ns/iter reference)
import jax
import jax.numpy as jnp
from jax import lax
from jax.experimental import pallas as pl
from jax.experimental.pallas import tpu as pltpu

N_DEV = 4


def kernel(A, B):
    m, _ = A.shape
    _, n = B.shape
    m_out = m // N_DEV

    def body(a_ref, b_ref, out_ref, p_ref, comm_ref, send_sems, recv_sems):
        my = lax.axis_index("i")
        left = lax.rem(my + N_DEV - 1, N_DEV)
        right = lax.rem(my + 1, N_DEV)

        barrier_sem = pltpu.get_barrier_semaphore()
        for nbr in (left, right):
            pl.semaphore_signal(
                barrier_sem, inc=1,
                device_id=(nbr,), device_id_type=pl.DeviceIdType.MESH,
            )
        pl.semaphore_wait(barrier_sem, 2)

        p_ref[:, :] = jnp.dot(
            a_ref[:, :], b_ref[:, :], preferred_element_type=jnp.float32
        )

        def chunk(c):
            return p_ref[pl.ds(c * m_out, m_out), :]

        comm_ref[0, :, :] = chunk(left)

        for s in range(N_DEV - 1):
            send_slot = s % 2
            recv_slot = (s + 1) % 2
            rdma = pltpu.make_async_remote_copy(
                src_ref=comm_ref.at[send_slot],
                dst_ref=comm_ref.at[recv_slot],
                send_sem=send_sems.at[send_slot],
                recv_sem=recv_sems.at[recv_slot],
                device_id=(right,),
                device_id_type=pl.DeviceIdType.MESH,
            )
            rdma.start()
            rdma.wait()

            c = lax.rem(my + N_DEV + 2 - s, N_DEV)
            if s < N_DEV - 2:
                comm_ref[recv_slot, :, :] = comm_ref[recv_slot, :, :] + chunk(c)
            else:
                out_ref[:, :] = comm_ref[recv_slot, :, :] + chunk(c)

    return pl.pallas_call(
        body,
        out_shape=jax.ShapeDtypeStruct((m_out, n), jnp.float32),
        in_specs=[
            pl.BlockSpec(memory_space=pltpu.VMEM),
            pl.BlockSpec(memory_space=pltpu.VMEM),
        ],
        out_specs=pl.BlockSpec(memory_space=pltpu.VMEM),
        scratch_shapes=[
            pltpu.VMEM((m, n), jnp.float32),
            pltpu.VMEM((2, m_out, n), jnp.float32),
            pltpu.SemaphoreType.DMA((2,)),
            pltpu.SemaphoreType.DMA((2,)),
        ],
        compiler_params=pltpu.CompilerParams(collective_id=0),
    )(A, B)


# device time: 88570 ns/iter; 1.8455x vs baseline; 1.8455x over previous
import jax
import jax.numpy as jnp
from jax import lax
from jax.experimental import pallas as pl
from jax.experimental.pallas import tpu as pltpu

N_DEV = 4


def kernel(A, B):
    m, _ = A.shape
    _, n = B.shape
    m_out = m // N_DEV
    n2 = n // 2

    def body(a_ref, b_ref, out_ref, p_ref,
             comm_cw, comm_ccw, send_cw, recv_cw, send_ccw, recv_ccw):
        my = lax.axis_index("i")
        left = lax.rem(my + N_DEV - 1, N_DEV)
        right = lax.rem(my + 1, N_DEV)

        barrier_sem = pltpu.get_barrier_semaphore()
        for nbr in (left, right):
            pl.semaphore_signal(
                barrier_sem, inc=1,
                device_id=(nbr,), device_id_type=pl.DeviceIdType.MESH,
            )
        pl.semaphore_wait(barrier_sem, 2)

        def a_rows(c):
            return a_ref[pl.ds(c * m_out, m_out), :]

        def compute_half(c, cw):
            if cw:
                p_ref[pl.ds(c * m_out, m_out), :n2] = jnp.dot(
                    a_rows(c), b_ref[:, :n2],
                    preferred_element_type=jnp.float32)
            else:
                p_ref[pl.ds(c * m_out, m_out), n2:] = jnp.dot(
                    a_rows(c), b_ref[:, n2:],
                    preferred_element_type=jnp.float32)

        def p_half(c, cw):
            if cw:
                return p_ref[pl.ds(c * m_out, m_out), :n2]
            return p_ref[pl.ds(c * m_out, m_out), n2:]

        def start_pair(s):
            send_slot = s % 2
            recv_slot = (s + 1) % 2
            r_cw = pltpu.make_async_remote_copy(
                src_ref=comm_cw.at[send_slot],
                dst_ref=comm_cw.at[recv_slot],
                send_sem=send_cw.at[send_slot],
                recv_sem=recv_cw.at[recv_slot],
                device_id=(right,),
                device_id_type=pl.DeviceIdType.MESH,
            )
            r_ccw = pltpu.make_async_remote_copy(
                src_ref=comm_ccw.at[send_slot],
                dst_ref=comm_ccw.at[recv_slot],
                send_sem=send_ccw.at[send_slot],
                recv_sem=recv_ccw.at[recv_slot],
                device_id=(left,),
                device_id_type=pl.DeviceIdType.MESH,
            )
            r_cw.start()
            r_ccw.start()
            return r_cw, r_ccw

        comm_cw[0, :, :] = jnp.dot(
            a_rows(left), b_ref[:, :n2], preferred_element_type=jnp.float32)
        comm_ccw[0, :, :] = jnp.dot(
            a_rows(right), b_ref[:, n2:], preferred_element_type=jnp.float32)
        r_cw, r_ccw = start_pair(0)

        opp = lax.rem(my + 2, N_DEV)
        compute_half(opp, True)
        compute_half(opp, False)
        compute_half(right, True)
        compute_half(left, False)
        compute_half(my, True)
        compute_half(my, False)

        for s in range(N_DEV - 1):
            r_cw.wait()
            r_ccw.wait()
            recv_slot = (s + 1) % 2
            c_cw = lax.rem(my + N_DEV + 2 - s, N_DEV)
            c_ccw = lax.rem(my + 2 + s, N_DEV)
            if s < N_DEV - 2:
                comm_cw[recv_slot, :, :] = (
                    comm_cw[recv_slot, :, :] + p_half(c_cw, True))
                comm_ccw[recv_slot, :, :] = (
                    comm_ccw[recv_slot, :, :] + p_half(c_ccw, False))
                r_cw, r_ccw = start_pair(s + 1)
            else:
                out_ref[:, :n2] = comm_cw[recv_slot, :, :] + p_half(c_cw, True)
                out_ref[:, n2:] = comm_ccw[recv_slot, :, :] + p_half(c_ccw, False)

    return pl.pallas_call(
        body,
        out_shape=jax.ShapeDtypeStruct((m_out, n), jnp.float32),
        in_specs=[
            pl.BlockSpec(memory_space=pltpu.VMEM),
            pl.BlockSpec(memory_space=pltpu.VMEM),
        ],
        out_specs=pl.BlockSpec(memory_space=pltpu.VMEM),
        scratch_shapes=[
            pltpu.VMEM((m, n), jnp.float32),
            pltpu.VMEM((2, m_out, n2), jnp.float32),
            pltpu.VMEM((2, m_out, n2), jnp.float32),
            pltpu.SemaphoreType.DMA((2,)),
            pltpu.SemaphoreType.DMA((2,)),
            pltpu.SemaphoreType.DMA((2,)),
            pltpu.SemaphoreType.DMA((2,)),
        ],
        compiler_params=pltpu.CompilerParams(collective_id=0),
    )(A, B)


# device time: 54579 ns/iter; 2.9948x vs baseline; 1.6228x over previous
import jax
import jax.numpy as jnp
from jax import lax
from jax.experimental import pallas as pl
from jax.experimental.pallas import tpu as pltpu

N_DEV = 4


def kernel(A, B):
    m, _ = A.shape
    _, n = B.shape
    m_out = m // N_DEV
    n2 = n // 2

    def body(a_ref, b_ref, out_ref, p_ref,
             comm_cw, comm_ccw, send_cw, recv_cw, send_ccw, recv_ccw):
        my = lax.axis_index("i")
        left = lax.rem(my + N_DEV - 1, N_DEV)
        right = lax.rem(my + 1, N_DEV)

        barrier_sem = pltpu.get_barrier_semaphore()
        for nbr in (left, right):
            pl.semaphore_signal(
                barrier_sem, inc=1,
                device_id=(nbr,), device_id_type=pl.DeviceIdType.MESH,
            )
        pl.semaphore_wait(barrier_sem, 2)

        def a_rows(c):
            return a_ref[pl.ds(c * m_out, m_out), :]

        def compute_half(c, cw):
            if cw:
                p_ref[pl.ds(c * m_out, m_out), :n2] = jnp.dot(
                    a_rows(c), b_ref[:, :n2],
                    preferred_element_type=jnp.float32)
            else:
                p_ref[pl.ds(c * m_out, m_out), n2:] = jnp.dot(
                    a_rows(c), b_ref[:, n2:],
                    preferred_element_type=jnp.float32)

        def p_half(c, cw):
            if cw:
                return p_ref[pl.ds(c * m_out, m_out), :n2]
            return p_ref[pl.ds(c * m_out, m_out), n2:]

        def start_pair(s):
            send_slot = s % 2
            recv_slot = (s + 1) % 2
            r_cw = pltpu.make_async_remote_copy(
                src_ref=comm_cw.at[send_slot],
                dst_ref=comm_cw.at[recv_slot],
                send_sem=send_cw.at[send_slot],
                recv_sem=recv_cw.at[recv_slot],
                device_id=(right,),
                device_id_type=pl.DeviceIdType.MESH,
            )
            r_ccw = pltpu.make_async_remote_copy(
                src_ref=comm_ccw.at[send_slot],
                dst_ref=comm_ccw.at[recv_slot],
                send_sem=send_ccw.at[send_slot],
                recv_sem=recv_ccw.at[recv_slot],
                device_id=(left,),
                device_id_type=pl.DeviceIdType.MESH,
            )
            r_cw.start()
            r_ccw.start()
            return r_cw, r_ccw

        comm_cw[0, :, :] = jnp.dot(
            a_rows(left), b_ref[:, :n2],
            preferred_element_type=jnp.float32).astype(jnp.bfloat16)
        comm_ccw[0, :, :] = jnp.dot(
            a_rows(right), b_ref[:, n2:],
            preferred_element_type=jnp.float32).astype(jnp.bfloat16)
        r_cw, r_ccw = start_pair(0)

        opp = lax.rem(my + 2, N_DEV)
        compute_half(opp, True)
        compute_half(opp, False)
        compute_half(right, True)
        compute_half(left, False)
        compute_half(my, True)
        compute_half(my, False)

        for s in range(N_DEV - 1):
            r_cw.wait()
            r_ccw.wait()
            recv_slot = (s + 1) % 2
            c_cw = lax.rem(my + N_DEV + 2 - s, N_DEV)
            c_ccw = lax.rem(my + 2 + s, N_DEV)
            if s < N_DEV - 2:
                comm_cw[recv_slot, :, :] = (
                    comm_cw[recv_slot, :, :].astype(jnp.float32)
                    + p_half(c_cw, True)).astype(jnp.bfloat16)
                comm_ccw[recv_slot, :, :] = (
                    comm_ccw[recv_slot, :, :].astype(jnp.float32)
                    + p_half(c_ccw, False)).astype(jnp.bfloat16)
                r_cw, r_ccw = start_pair(s + 1)
            else:
                out_ref[:, :n2] = (
                    comm_cw[recv_slot, :, :].astype(jnp.float32)
                    + p_half(c_cw, True))
                out_ref[:, n2:] = (
                    comm_ccw[recv_slot, :, :].astype(jnp.float32)
                    + p_half(c_ccw, False))

    return pl.pallas_call(
        body,
        out_shape=jax.ShapeDtypeStruct((m_out, n), jnp.float32),
        in_specs=[
            pl.BlockSpec(memory_space=pltpu.VMEM),
            pl.BlockSpec(memory_space=pltpu.VMEM),
        ],
        out_specs=pl.BlockSpec(memory_space=pltpu.VMEM),
        scratch_shapes=[
            pltpu.VMEM((m, n), jnp.float32),
            pltpu.VMEM((2, m_out, n2), jnp.bfloat16),
            pltpu.VMEM((2, m_out, n2), jnp.bfloat16),
            pltpu.SemaphoreType.DMA((2,)),
            pltpu.SemaphoreType.DMA((2,)),
            pltpu.SemaphoreType.DMA((2,)),
            pltpu.SemaphoreType.DMA((2,)),
        ],
        compiler_params=pltpu.CompilerParams(collective_id=0),
    )(A, B)


# device time: 50498 ns/iter; 3.2368x vs baseline; 1.0808x over previous
import jax
import jax.numpy as jnp
from jax import lax
from jax.experimental import pallas as pl
from jax.experimental.pallas import tpu as pltpu

N_DEV = 4


def kernel(A, B):
    m, _ = A.shape
    _, n = B.shape
    m_out = m // N_DEV
    n2 = n // 2

    def body(a_ref, b_ref, out_ref, p_ref,
             comm_cw, comm_ccw, send_cw, recv_cw, send_ccw, recv_ccw):
        my = lax.axis_index("i")
        left = lax.rem(my + N_DEV - 1, N_DEV)
        right = lax.rem(my + 1, N_DEV)

        barrier_sem = pltpu.get_barrier_semaphore()
        for nbr in (left, right):
            pl.semaphore_signal(
                barrier_sem, inc=1,
                device_id=(nbr,), device_id_type=pl.DeviceIdType.MESH,
            )
        pl.semaphore_wait(barrier_sem, 2)

        def a_rows(c):
            return a_ref[pl.ds(c * m_out, m_out), :]

        def compute_half(c, cw):
            if cw:
                p_ref[pl.ds(c * m_out, m_out), :n2] = jnp.dot(
                    a_rows(c), b_ref[:, :n2],
                    preferred_element_type=jnp.float32)
            else:
                p_ref[pl.ds(c * m_out, m_out), n2:] = jnp.dot(
                    a_rows(c), b_ref[:, n2:],
                    preferred_element_type=jnp.float32)

        m_seg = m_out // 2

        def seg_rows(seg):
            return pl.ds(seg * m_seg, m_seg)

        def make_seg(s, seg, cw):
            send_slot = s % 2
            recv_slot = (s + 1) % 2
            comm = comm_cw if cw else comm_ccw
            ssem = send_cw if cw else send_ccw
            rsem = recv_cw if cw else recv_ccw
            return pltpu.make_async_remote_copy(
                src_ref=comm.at[send_slot, seg_rows(seg), :],
                dst_ref=comm.at[recv_slot, seg_rows(seg), :],
                send_sem=ssem.at[send_slot, seg],
                recv_sem=rsem.at[recv_slot, seg],
                device_id=(right if cw else left,),
                device_id_type=pl.DeviceIdType.MESH,
            )

        comm_cw[0, :, :] = jnp.dot(
            a_rows(left), b_ref[:, :n2],
            preferred_element_type=jnp.float32).astype(jnp.bfloat16)
        r_cw = [make_seg(0, seg, True) for seg in range(2)]
        for r in r_cw:
            r.start()
        comm_ccw[0, :, :] = jnp.dot(
            a_rows(right), b_ref[:, n2:],
            preferred_element_type=jnp.float32).astype(jnp.bfloat16)
        r_ccw = [make_seg(0, seg, False) for seg in range(2)]
        for r in r_ccw:
            r.start()

        opp = lax.rem(my + 2, N_DEV)
        compute_half(opp, True)
        compute_half(opp, False)
        compute_half(right, True)
        compute_half(left, False)
        compute_half(my, True)
        compute_half(my, False)

        for s in range(N_DEV - 1):
            recv_slot = (s + 1) % 2
            c_cw = lax.rem(my + N_DEV + 2 - s, N_DEV)
            c_ccw = lax.rem(my + 2 + s, N_DEV)
            next_cw, next_ccw = [], []
            for seg in range(2):
                for cw in (True, False):
                    r = (r_cw if cw else r_ccw)[seg]
                    r.wait()
                    comm = comm_cw if cw else comm_ccw
                    c = c_cw if cw else c_ccw
                    if cw:
                        p_seg = p_ref[pl.ds(c * m_out + seg * m_seg, m_seg), :n2]
                    else:
                        p_seg = p_ref[pl.ds(c * m_out + seg * m_seg, m_seg), n2:]
                    if s < N_DEV - 2:
                        comm[recv_slot, seg_rows(seg), :] = (
                            comm[recv_slot, seg_rows(seg), :].astype(jnp.float32)
                            + p_seg).astype(jnp.bfloat16)
                        nr = make_seg(s + 1, seg, cw)
                        nr.start()
                        (next_cw if cw else next_ccw).append(nr)
                    else:
                        if cw:
                            out_ref[seg_rows(seg), :n2] = (
                                comm[recv_slot, seg_rows(seg), :].astype(
                                    jnp.float32) + p_seg)
                        else:
                            out_ref[seg_rows(seg), n2:] = (
                                comm[recv_slot, seg_rows(seg), :].astype(
                                    jnp.float32) + p_seg)
            r_cw, r_ccw = next_cw, next_ccw

    return pl.pallas_call(
        body,
        out_shape=jax.ShapeDtypeStruct((m_out, n), jnp.float32),
        in_specs=[
            pl.BlockSpec(memory_space=pltpu.VMEM),
            pl.BlockSpec(memory_space=pltpu.VMEM),
        ],
        out_specs=pl.BlockSpec(memory_space=pltpu.VMEM),
        scratch_shapes=[
            pltpu.VMEM((m, n), jnp.float32),
            pltpu.VMEM((2, m_out, n2), jnp.bfloat16),
            pltpu.VMEM((2, m_out, n2), jnp.bfloat16),
            pltpu.SemaphoreType.DMA((2, 2)),
            pltpu.SemaphoreType.DMA((2, 2)),
            pltpu.SemaphoreType.DMA((2, 2)),
            pltpu.SemaphoreType.DMA((2, 2)),
        ],
        compiler_params=pltpu.CompilerParams(collective_id=0),
    )(A, B)


# device time: 48857 ns/iter; 3.3455x vs baseline; 1.0336x over previous
import jax
import jax.numpy as jnp
from jax import lax
from jax.experimental import pallas as pl
from jax.experimental.pallas import tpu as pltpu

N_DEV = 4


def kernel(A, B):
    m, _ = A.shape
    _, n = B.shape
    m_out = m // N_DEV
    n2 = n // 2

    def body(a_ref, b_ref, out_ref, p_ref,
             comm_cw, comm_ccw, send_cw, recv_cw, send_ccw, recv_ccw):
        my = lax.axis_index("i")
        left = lax.rem(my + N_DEV - 1, N_DEV)
        right = lax.rem(my + 1, N_DEV)

        barrier_sem = pltpu.get_barrier_semaphore()
        for nbr in (left, right):
            pl.semaphore_signal(
                barrier_sem, inc=1,
                device_id=(nbr,), device_id_type=pl.DeviceIdType.MESH,
            )
        pl.semaphore_wait(barrier_sem, 2)

        def a_rows(c):
            return a_ref[pl.ds(c * m_out, m_out), :]

        def compute_half(c, cw):
            if cw:
                p_ref[pl.ds(c * m_out, m_out), :n2] = jnp.dot(
                    a_rows(c), b_ref[:, :n2],
                    preferred_element_type=jnp.float32)
            else:
                p_ref[pl.ds(c * m_out, m_out), n2:] = jnp.dot(
                    a_rows(c), b_ref[:, n2:],
                    preferred_element_type=jnp.float32)

        n_seg = 4
        m_seg = m_out // n_seg

        def seg_rows(seg):
            return pl.ds(seg * m_seg, m_seg)

        def make_seg(s, seg, cw):
            send_slot = s % 2
            recv_slot = (s + 1) % 2
            comm = comm_cw if cw else comm_ccw
            ssem = send_cw if cw else send_ccw
            rsem = recv_cw if cw else recv_ccw
            return pltpu.make_async_remote_copy(
                src_ref=comm.at[send_slot, seg_rows(seg), :],
                dst_ref=comm.at[recv_slot, seg_rows(seg), :],
                send_sem=ssem.at[send_slot, seg],
                recv_sem=rsem.at[recv_slot, seg],
                device_id=(right if cw else left,),
                device_id_type=pl.DeviceIdType.MESH,
            )

        r_cw, r_ccw = [], []
        for seg in range(n_seg):
            comm_cw[0, seg_rows(seg), :] = jnp.dot(
                a_ref[pl.ds(left * m_out + seg * m_seg, m_seg), :],
                b_ref[:, :n2],
                preferred_element_type=jnp.float32).astype(jnp.bfloat16)
            r = make_seg(0, seg, True)
            r.start()
            r_cw.append(r)
            comm_ccw[0, seg_rows(seg), :] = jnp.dot(
                a_ref[pl.ds(right * m_out + seg * m_seg, m_seg), :],
                b_ref[:, n2:],
                preferred_element_type=jnp.float32).astype(jnp.bfloat16)
            r = make_seg(0, seg, False)
            r.start()
            r_ccw.append(r)

        opp = lax.rem(my + 2, N_DEV)
        compute_half(opp, True)
        compute_half(opp, False)
        compute_half(right, True)
        compute_half(left, False)
        compute_half(my, True)
        compute_half(my, False)

        for s in range(N_DEV - 1):
            recv_slot = (s + 1) % 2
            c_cw = lax.rem(my + N_DEV + 2 - s, N_DEV)
            c_ccw = lax.rem(my + 2 + s, N_DEV)
            next_cw, next_ccw = [], []
            for seg in range(n_seg):
                for cw in (True, False):
                    r = (r_cw if cw else r_ccw)[seg]
                    r.wait()
                    comm = comm_cw if cw else comm_ccw
                    c = c_cw if cw else c_ccw
                    if cw:
                        p_seg = p_ref[pl.ds(c * m_out + seg * m_seg, m_seg), :n2]
                    else:
                        p_seg = p_ref[pl.ds(c * m_out + seg * m_seg, m_seg), n2:]
                    if s < N_DEV - 2:
                        comm[recv_slot, seg_rows(seg), :] = (
                            comm[recv_slot, seg_rows(seg), :].astype(jnp.float32)
                            + p_seg).astype(jnp.bfloat16)
                        nr = make_seg(s + 1, seg, cw)
                        nr.start()
                        (next_cw if cw else next_ccw).append(nr)
                    else:
                        if cw:
                            out_ref[seg_rows(seg), :n2] = (
                                comm[recv_slot, seg_rows(seg), :].astype(
                                    jnp.float32) + p_seg)
                        else:
                            out_ref[seg_rows(seg), n2:] = (
                                comm[recv_slot, seg_rows(seg), :].astype(
                                    jnp.float32) + p_seg)
            r_cw, r_ccw = next_cw, next_ccw

    return pl.pallas_call(
        body,
        out_shape=jax.ShapeDtypeStruct((m_out, n), jnp.float32),
        in_specs=[
            pl.BlockSpec(memory_space=pltpu.VMEM),
            pl.BlockSpec(memory_space=pltpu.VMEM),
        ],
        out_specs=pl.BlockSpec(memory_space=pltpu.VMEM),
        scratch_shapes=[
            pltpu.VMEM((m, n), jnp.float32),
            pltpu.VMEM((2, m_out, n2), jnp.bfloat16),
            pltpu.VMEM((2, m_out, n2), jnp.bfloat16),
            pltpu.SemaphoreType.DMA((2, 4)),
            pltpu.SemaphoreType.DMA((2, 4)),
            pltpu.SemaphoreType.DMA((2, 4)),
            pltpu.SemaphoreType.DMA((2, 4)),
        ],
        compiler_params=pltpu.CompilerParams(collective_id=0),
    )(A, B)


# device time: 48809 ns/iter; 3.3488x vs baseline; 1.0010x over previous
import jax
import jax.numpy as jnp
from jax import lax
from jax.experimental import pallas as pl
from jax.experimental.pallas import tpu as pltpu

N_DEV = 4


def kernel(A, B):
    m, _ = A.shape
    _, n = B.shape
    m_out = m // N_DEV
    n2 = n // 2

    def body(a_ref, b_ref, out_ref, p_ref,
             comm_cw, comm_ccw, send_cw, recv_cw, send_ccw, recv_ccw):
        my = lax.axis_index("i")
        left = lax.rem(my + N_DEV - 1, N_DEV)
        right = lax.rem(my + 1, N_DEV)

        barrier_sem = pltpu.get_barrier_semaphore()
        for nbr in (left, right):
            pl.semaphore_signal(
                barrier_sem, inc=1,
                device_id=(nbr,), device_id_type=pl.DeviceIdType.MESH,
            )
        pl.semaphore_wait(barrier_sem, 2)

        def a_rows(c):
            return a_ref[pl.ds(c * m_out, m_out), :]

        def compute_half(c, cw):
            if cw:
                p_ref[pl.ds(c * m_out, m_out), :n2] = jnp.dot(
                    a_rows(c), b_ref[:, :n2],
                    preferred_element_type=jnp.float32)
            else:
                p_ref[pl.ds(c * m_out, m_out), n2:] = jnp.dot(
                    a_rows(c), b_ref[:, n2:],
                    preferred_element_type=jnp.float32)

        n_seg = 4
        m_seg = m_out // n_seg

        def seg_rows(seg):
            return pl.ds(seg * m_seg, m_seg)

        def make_seg(s, seg, cw):
            send_slot = s % 2
            recv_slot = (s + 1) % 2
            comm = comm_cw if cw else comm_ccw
            ssem = send_cw if cw else send_ccw
            rsem = recv_cw if cw else recv_ccw
            return pltpu.make_async_remote_copy(
                src_ref=comm.at[send_slot, seg_rows(seg), :],
                dst_ref=comm.at[recv_slot, seg_rows(seg), :],
                send_sem=ssem.at[send_slot, seg],
                recv_sem=rsem.at[recv_slot, seg],
                device_id=(right if cw else left,),
                device_id_type=pl.DeviceIdType.MESH,
            )

        r_cw, r_ccw = [], []
        for seg in range(n_seg):
            comm_cw[0, seg_rows(seg), :] = jnp.dot(
                a_ref[pl.ds(left * m_out + seg * m_seg, m_seg), :],
                b_ref[:, :n2],
                preferred_element_type=jnp.float32).astype(jnp.bfloat16)
            r = make_seg(0, seg, True)
            r.start()
            r_cw.append(r)
            comm_ccw[0, seg_rows(seg), :] = jnp.dot(
                a_ref[pl.ds(right * m_out + seg * m_seg, m_seg), :],
                b_ref[:, n2:],
                preferred_element_type=jnp.float32).astype(jnp.bfloat16)
            r = make_seg(0, seg, False)
            r.start()
            r_ccw.append(r)

        opp = lax.rem(my + 2, N_DEV)
        hop_chunks = [(opp, opp), (right, left), (my, my)]

        for s in range(N_DEV - 1):
            cc, cc2 = hop_chunks[s]
            compute_half(cc, True)
            compute_half(cc2, False)
            recv_slot = (s + 1) % 2
            c_cw = lax.rem(my + N_DEV + 2 - s, N_DEV)
            c_ccw = lax.rem(my + 2 + s, N_DEV)
            next_cw, next_ccw = [], []
            for seg in range(n_seg):
                for cw in (True, False):
                    r = (r_cw if cw else r_ccw)[seg]
                    r.wait()
                    comm = comm_cw if cw else comm_ccw
                    c = c_cw if cw else c_ccw
                    if cw:
                        p_seg = p_ref[pl.ds(c * m_out + seg * m_seg, m_seg), :n2]
                    else:
                        p_seg = p_ref[pl.ds(c * m_out + seg * m_seg, m_seg), n2:]
                    if s < N_DEV - 2:
                        comm[recv_slot, seg_rows(seg), :] = (
                            comm[recv_slot, seg_rows(seg), :].astype(jnp.float32)
                            + p_seg).astype(jnp.bfloat16)
                        nr = make_seg(s + 1, seg, cw)
                        nr.start()
                        (next_cw if cw else next_ccw).append(nr)
                    else:
                        if cw:
                            out_ref[seg_rows(seg), :n2] = (
                                comm[recv_slot, seg_rows(seg), :].astype(
                                    jnp.float32) + p_seg)
                        else:
                            out_ref[seg_rows(seg), n2:] = (
                                comm[recv_slot, seg_rows(seg), :].astype(
                                    jnp.float32) + p_seg)
            r_cw, r_ccw = next_cw, next_ccw

    return pl.pallas_call(
        body,
        out_shape=jax.ShapeDtypeStruct((m_out, n), jnp.float32),
        in_specs=[
            pl.BlockSpec(memory_space=pltpu.VMEM),
            pl.BlockSpec(memory_space=pltpu.VMEM),
        ],
        out_specs=pl.BlockSpec(memory_space=pltpu.VMEM),
        scratch_shapes=[
            pltpu.VMEM((m, n), jnp.float32),
            pltpu.VMEM((2, m_out, n2), jnp.bfloat16),
            pltpu.VMEM((2, m_out, n2), jnp.bfloat16),
            pltpu.SemaphoreType.DMA((2, 4)),
            pltpu.SemaphoreType.DMA((2, 4)),
            pltpu.SemaphoreType.DMA((2, 4)),
            pltpu.SemaphoreType.DMA((2, 4)),
        ],
        compiler_params=pltpu.CompilerParams(collective_id=0),
    )(A, B)
